# baseline (speedup 1.0000x reference)
import jax
import jax.numpy as jnp
import numpy as np
from jax.experimental import pallas as pl
from jax.experimental.pallas import tpu as pltpu

F32 = jnp.float32
BF16 = jnp.bfloat16

S = 2048
D = 768
E = 8
DFF = 3072
NF = S // 2 + 1
FP = 1152
T = 128
NB = 16
NR = FP // T


def _proj_router_kernel(xfr_ref, wpin_ref, bpin_ref, eidx_ref,
                        xp_ref, pos_ref, bexp_ref, bvalid_ref):
    xfr = xfr_ref[...].astype(BF16)
    xp = jax.lax.dot_general(
        xfr, wpin_ref[...].astype(BF16), (((1,), (1,)), ((), ())),
        preferred_element_type=F32) + bpin_ref[...]
    xp_ref[...] = xp.astype(BF16)
    eidx = eidx_ref[...]
    e_iota = jax.lax.broadcasted_iota(jnp.int32, (FP, E), 1)
    M = (e_iota == eidx).astype(F32)
    t_iota = jax.lax.broadcasted_iota(jnp.int32, (FP, 1), 0)
    validm = (t_iota < NF).astype(F32)
    Mv = M * validm
    r_i = jax.lax.broadcasted_iota(jnp.int32, (FP, FP), 0)
    c_i = jax.lax.broadcasted_iota(jnp.int32, (FP, FP), 1)
    slt = (c_i < r_i).astype(BF16)
    ranks = jax.lax.dot_general(slt, Mv.astype(BF16), (((1,), (0,)), ((), ())),
                                preferred_element_type=F32)
    rank_pt = jnp.sum(M * ranks, axis=1, keepdims=True)
    counts = jnp.sum(Mv, axis=0, keepdims=True).astype(jnp.int32)
    cumblk = jnp.zeros((), jnp.int32)
    base_f = jnp.zeros((FP, 1), F32)
    cums = []
    for e in range(E):
        base_f = base_f + M[:, e:e + 1] * (cumblk * T).astype(F32)
        cumblk = cumblk + (counts[0, e] + T - 1) // T
        cums.append(cumblk)
    pos = base_f + rank_pt
    pos = jnp.where(validm > 0.0, pos, 4000.0).astype(jnp.int32)
    pos_ref[...] = pos
    b_iota = jax.lax.broadcasted_iota(jnp.int32, (1, NB), 1)
    be = jnp.zeros((1, NB), jnp.int32)
    for e in range(E):
        be = be + (b_iota >= cums[e]).astype(jnp.int32)
    bexp_ref[...] = jnp.minimum(be, E - 1)
    bvalid_ref[...] = (b_iota < cumblk).astype(jnp.int32)


def _moe_kernel(bexp_ref, bvalid_ref, pos_ref, xp_ref, w1_ref, b1_ref,
                w2_ref, b2_ref, out_ref):
    b = pl.program_id(0)
    j = pl.program_id(1)

    @pl.when((b == 0) & (j == 0))
    def _init():
        out_ref[...] = jnp.zeros_like(out_ref)

    @pl.when(bvalid_ref[b] > 0)
    def _work():
        slot_i = jax.lax.broadcasted_iota(jnp.int32, (T, T), 0) + b * T
        oh = []
        xg = jnp.zeros((T, D), F32)
        for r in range(NR):
            ohr = (pos_ref[r:r + 1, :] == slot_i).astype(BF16)
            xg = xg + jax.lax.dot_general(
                ohr, xp_ref[r * T:(r + 1) * T, :], (((1,), (0,)), ((), ())),
                preferred_element_type=F32)
            oh.append(ohr)
        h = jax.lax.dot_general(
            xg.astype(BF16), w1_ref[0].astype(BF16), (((1,), (1,)), ((), ())),
            preferred_element_type=F32) + b1_ref[0]
        h = 0.5 * h * (1.0 + jax.lax.erf(h * 0.7071067811865476))
        eo = jax.lax.dot_general(
            h.astype(BF16), w2_ref[0].astype(BF16), (((1,), (1,)), ((), ())),
            preferred_element_type=F32) + jnp.where(j == 0, 1.0, 0.0) * b2_ref[0]
        eob = eo.astype(BF16)
        for r in range(NR):
            out_ref[r * T:(r + 1) * T, :] += jax.lax.dot_general(
                oh[r], eob, (((0,), (0,)), ((), ())),
                preferred_element_type=F32)


def _proj_out_kernel(o_ref, wpo_ref, bpo_ref, y_ref):
    y_ref[...] = (jax.lax.dot_general(
        o_ref[...].astype(BF16), wpo_ref[...].astype(BF16),
        (((1,), (1,)), ((), ())),
        preferred_element_type=F32) + bpo_ref[...]).astype(BF16)


def _idft_ln_kernel(op_ref, fc_ref, fs_ref, x_ref, g_ref, bt_ref, y_ref):
    opm = op_ref[...]
    ot = (jax.lax.dot_general(fc_ref[...], opm[:, :D],
                              (((1,), (0,)), ((), ())),
                              preferred_element_type=F32)
          + jax.lax.dot_general(fs_ref[...], opm[:, D:],
                                (((1,), (0,)), ((), ())),
                                preferred_element_type=F32))
    mu = jnp.mean(ot, axis=1, keepdims=True)
    var = jnp.mean((ot - mu) ** 2, axis=1, keepdims=True)
    y_ref[...] = ((ot - mu) * jax.lax.rsqrt(var + 1e-5) * g_ref[...]
                  + bt_ref[...] + x_ref[...])


def _build_idft_mats():
    k = np.arange(FP)
    s = np.arange(S)
    ang = (2.0 * np.pi / S) * np.outer(s, k)
    ck = np.where(k == 0, 1.0, 2.0)
    ck[S // 2] = 1.0
    ck[k > S // 2] = 0.0
    fc = (ck / np.sqrt(S)) * np.cos(ang)
    cs = np.where((k == 0) | (k >= S // 2), 0.0, 2.0)
    fs = -(cs / np.sqrt(S)) * np.sin(ang)
    return fc.astype(np.float32), fs.astype(np.float32)


_FC, _FS = _build_idft_mats()


@jax.jit
def kernel(x, Wp_in, bp_in, Wr, br, Ew1, Eb1, Ew2, Eb2, Wp_out, bp_out,
           gamma, beta):
    xf = jnp.fft.rfft(x, axis=1, norm='ortho')
    xfr3 = jnp.concatenate([xf.real, xf.imag], axis=-1)
    xfr_p = jnp.pad(xfr3[0], ((0, FP - NF), (0, 0)))

    xp_route = xfr3 @ Wp_in.T + bp_in
    logits = xp_route @ Wr.T + br
    ti = jnp.argmax(logits, axis=-1)
    eidx = jnp.pad(ti[0].astype(jnp.int32), (0, FP - NF),
                   constant_values=E).reshape(FP, 1)

    xp, pos, bexp, bvalid = pl.pallas_call(
        _proj_router_kernel,
        out_shape=[
            jax.ShapeDtypeStruct((FP, D), BF16),
            jax.ShapeDtypeStruct((FP, 1), jnp.int32),
            jax.ShapeDtypeStruct((1, NB), jnp.int32),
            jax.ShapeDtypeStruct((1, NB), jnp.int32),
        ],
    )(xfr_p, Wp_in, bp_in.reshape(1, D), eidx)

    pos9 = pos.reshape(NR, T)
    DH = DFF // 2
    grid_spec = pltpu.PrefetchScalarGridSpec(
        num_scalar_prefetch=2,
        grid=(NB, 2),
        in_specs=[
            pl.BlockSpec((NR, T), lambda b, j, bexp, bvalid: (0, 0)),
            pl.BlockSpec((FP, D), lambda b, j, bexp, bvalid: (0, 0)),
            pl.BlockSpec((1, DH, D),
                         lambda b, j, bexp, bvalid: (bexp[b], j, 0)),
            pl.BlockSpec((1, 1, DH),
                         lambda b, j, bexp, bvalid: (bexp[b], 0, j)),
            pl.BlockSpec((1, D, DH),
                         lambda b, j, bexp, bvalid: (bexp[b], 0, j)),
            pl.BlockSpec((1, 1, D),
                         lambda b, j, bexp, bvalid: (bexp[b], 0, 0)),
        ],
        out_specs=pl.BlockSpec((FP, D), lambda b, j, bexp, bvalid: (0, 0)),
    )
    out = pl.pallas_call(
        _moe_kernel,
        grid_spec=grid_spec,
        out_shape=jax.ShapeDtypeStruct((FP, D), F32),
    )(bexp.reshape(NB), bvalid.reshape(NB), pos9, xp, Ew1,
      Eb1.reshape(E, 1, DFF), Ew2, Eb2.reshape(E, 1, D))

    op = pl.pallas_call(
        _proj_out_kernel,
        out_shape=jax.ShapeDtypeStruct((FP, 2 * D), BF16),
    )(out, Wp_out, bp_out.reshape(1, 2 * D))

    RB = 512
    y = pl.pallas_call(
        _idft_ln_kernel,
        grid=(S // RB,),
        in_specs=[
            pl.BlockSpec((FP, 2 * D), lambda i: (0, 0)),
            pl.BlockSpec((RB, FP), lambda i: (i, 0)),
            pl.BlockSpec((RB, FP), lambda i: (i, 0)),
            pl.BlockSpec((RB, D), lambda i: (i, 0)),
            pl.BlockSpec((1, D), lambda i: (0, 0)),
            pl.BlockSpec((1, D), lambda i: (0, 0)),
        ],
        out_specs=pl.BlockSpec((RB, D), lambda i: (i, 0)),
        out_shape=jax.ShapeDtypeStruct((S, D), F32),
    )(op, jnp.asarray(_FC, dtype=BF16), jnp.asarray(_FS, dtype=BF16), x[0],
      gamma.reshape(1, D), beta.reshape(1, D))
    return y[None]


# speedup vs baseline: 1.1514x; 1.1514x over previous
import jax
import jax.numpy as jnp
import numpy as np
from jax.experimental import pallas as pl
from jax.experimental.pallas import tpu as pltpu

F32 = jnp.float32
BF16 = jnp.bfloat16

S = 2048
D = 768
E = 8
DFF = 3072
NF = S // 2 + 1
FP = 1152
T = 128
NB = 16
NR = FP // T


def _proj_router_kernel(xfr_ref, wpin_ref, bpin_ref, eidx_ref,
                        xp_ref, pos_ref, bexp_ref, bvalid_ref):
    xfr = xfr_ref[...].astype(BF16)
    xp = jax.lax.dot_general(
        xfr, wpin_ref[...].astype(BF16), (((1,), (1,)), ((), ())),
        preferred_element_type=F32) + bpin_ref[...]
    xp_ref[...] = xp.astype(BF16)
    eidx = eidx_ref[...]
    e_iota = jax.lax.broadcasted_iota(jnp.int32, (FP, E), 1)
    M = (e_iota == eidx).astype(F32)
    t_iota = jax.lax.broadcasted_iota(jnp.int32, (FP, 1), 0)
    validm = (t_iota < NF).astype(F32)
    Mv = M * validm
    r_i = jax.lax.broadcasted_iota(jnp.int32, (FP, FP), 0)
    c_i = jax.lax.broadcasted_iota(jnp.int32, (FP, FP), 1)
    slt = (c_i < r_i).astype(BF16)
    ranks = jax.lax.dot_general(slt, Mv.astype(BF16), (((1,), (0,)), ((), ())),
                                preferred_element_type=F32)
    rank_pt = jnp.sum(M * ranks, axis=1, keepdims=True)
    counts = jnp.sum(Mv, axis=0, keepdims=True).astype(jnp.int32)
    cumblk = jnp.zeros((), jnp.int32)
    base_f = jnp.zeros((FP, 1), F32)
    cums = []
    for e in range(E):
        base_f = base_f + M[:, e:e + 1] * (cumblk * T).astype(F32)
        cumblk = cumblk + (counts[0, e] + T - 1) // T
        cums.append(cumblk)
    pos = base_f + rank_pt
    pos = jnp.where(validm > 0.0, pos, 4000.0).astype(jnp.int32)
    pos_ref[...] = pos
    b_iota = jax.lax.broadcasted_iota(jnp.int32, (1, NB), 1)
    be = jnp.zeros((1, NB), jnp.int32)
    for e in range(E):
        be = be + (b_iota >= cums[e]).astype(jnp.int32)
    bexp_ref[...] = jnp.minimum(be, E - 1)
    bvalid_ref[...] = (b_iota < cumblk).astype(jnp.int32)


def _moe_kernel(bexp_ref, bvalid_ref, pos_ref, xp_ref, w1_ref, b1_ref,
                w2_ref, b2_ref, out_ref):
    b = pl.program_id(0)

    @pl.when(b == 0)
    def _init():
        out_ref[...] = jnp.zeros_like(out_ref)

    @pl.when(bvalid_ref[b] > 0)
    def _work():
        slot_i = jax.lax.broadcasted_iota(jnp.int32, (T, T), 0) + b * T
        oh = []
        xg = jnp.zeros((T, D), F32)
        for r in range(NR):
            ohr = (pos_ref[r:r + 1, :] == slot_i).astype(BF16)
            xg = xg + jax.lax.dot_general(
                ohr, xp_ref[r * T:(r + 1) * T, :], (((1,), (0,)), ((), ())),
                preferred_element_type=F32)
            oh.append(ohr)
        h = jax.lax.dot_general(
            xg.astype(BF16), w1_ref[0].astype(BF16), (((1,), (1,)), ((), ())),
            preferred_element_type=F32) + b1_ref[0]
        h = 0.5 * h * (1.0 + jax.lax.erf(h * 0.7071067811865476))
        eo = jax.lax.dot_general(
            h.astype(BF16), w2_ref[0].astype(BF16), (((1,), (1,)), ((), ())),
            preferred_element_type=F32) + b2_ref[0]
        eob = eo.astype(BF16)
        for r in range(NR):
            out_ref[r * T:(r + 1) * T, :] += jax.lax.dot_general(
                oh[r], eob, (((0,), (0,)), ((), ())),
                preferred_element_type=F32)


def _proj_out_kernel(o_ref, wpo_ref, bpo_ref, y_ref):
    y_ref[...] = (jax.lax.dot_general(
        o_ref[...].astype(BF16), wpo_ref[...].astype(BF16),
        (((1,), (1,)), ((), ())),
        preferred_element_type=F32) + bpo_ref[...]).astype(BF16)


def _idft_ln_kernel(op_ref, fc_ref, fs_ref, x_ref, g_ref, bt_ref, y_ref):
    opm = op_ref[...]
    ot = (jax.lax.dot_general(fc_ref[...], opm[:, :D],
                              (((1,), (0,)), ((), ())),
                              preferred_element_type=F32)
          + jax.lax.dot_general(fs_ref[...], opm[:, D:],
                                (((1,), (0,)), ((), ())),
                                preferred_element_type=F32))
    mu = jnp.mean(ot, axis=1, keepdims=True)
    var = jnp.mean((ot - mu) ** 2, axis=1, keepdims=True)
    y_ref[...] = ((ot - mu) * jax.lax.rsqrt(var + 1e-5) * g_ref[...]
                  + bt_ref[...] + x_ref[...])


def _build_idft_mats():
    k = np.arange(FP)
    s = np.arange(S)
    ang = (2.0 * np.pi / S) * np.outer(s, k)
    ck = np.where(k == 0, 1.0, 2.0)
    ck[S // 2] = 1.0
    ck[k > S // 2] = 0.0
    fc = (ck / np.sqrt(S)) * np.cos(ang)
    cs = np.where((k == 0) | (k >= S // 2), 0.0, 2.0)
    fs = -(cs / np.sqrt(S)) * np.sin(ang)
    return fc.astype(np.float32), fs.astype(np.float32)


_FC, _FS = _build_idft_mats()


@jax.jit
def kernel(x, Wp_in, bp_in, Wr, br, Ew1, Eb1, Ew2, Eb2, Wp_out, bp_out,
           gamma, beta):
    xf = jnp.fft.rfft(x, axis=1, norm='ortho')
    xfr3 = jnp.concatenate([xf.real, xf.imag], axis=-1)
    xfr_p = jnp.pad(xfr3[0], ((0, FP - NF), (0, 0)))

    xp_route = xfr3 @ Wp_in.T + bp_in
    logits = xp_route @ Wr.T + br
    ti = jnp.argmax(logits, axis=-1)
    eidx = jnp.pad(ti[0].astype(jnp.int32), (0, FP - NF),
                   constant_values=E).reshape(FP, 1)

    xp, pos, bexp, bvalid = pl.pallas_call(
        _proj_router_kernel,
        out_shape=[
            jax.ShapeDtypeStruct((FP, D), BF16),
            jax.ShapeDtypeStruct((FP, 1), jnp.int32),
            jax.ShapeDtypeStruct((1, NB), jnp.int32),
            jax.ShapeDtypeStruct((1, NB), jnp.int32),
        ],
    )(xfr_p, Wp_in, bp_in.reshape(1, D), eidx)

    pos9 = pos.reshape(NR, T)
    grid_spec = pltpu.PrefetchScalarGridSpec(
        num_scalar_prefetch=2,
        grid=(NB,),
        in_specs=[
            pl.BlockSpec((NR, T), lambda b, bexp, bvalid: (0, 0)),
            pl.BlockSpec((FP, D), lambda b, bexp, bvalid: (0, 0)),
            pl.BlockSpec((1, DFF, D), lambda b, bexp, bvalid: (bexp[b], 0, 0)),
            pl.BlockSpec((1, 1, DFF), lambda b, bexp, bvalid: (bexp[b], 0, 0)),
            pl.BlockSpec((1, D, DFF), lambda b, bexp, bvalid: (bexp[b], 0, 0)),
            pl.BlockSpec((1, 1, D), lambda b, bexp, bvalid: (bexp[b], 0, 0)),
        ],
        out_specs=pl.BlockSpec((FP, D), lambda b, bexp, bvalid: (0, 0)),
    )
    out = pl.pallas_call(
        _moe_kernel,
        grid_spec=grid_spec,
        out_shape=jax.ShapeDtypeStruct((FP, D), F32),
    )(bexp.reshape(NB), bvalid.reshape(NB), pos9, xp, Ew1,
      Eb1.reshape(E, 1, DFF), Ew2, Eb2.reshape(E, 1, D))

    op = pl.pallas_call(
        _proj_out_kernel,
        out_shape=jax.ShapeDtypeStruct((FP, 2 * D), BF16),
    )(out, Wp_out, bp_out.reshape(1, 2 * D))

    RB = 512
    y = pl.pallas_call(
        _idft_ln_kernel,
        grid=(S // RB,),
        in_specs=[
            pl.BlockSpec((FP, 2 * D), lambda i: (0, 0)),
            pl.BlockSpec((RB, FP), lambda i: (i, 0)),
            pl.BlockSpec((RB, FP), lambda i: (i, 0)),
            pl.BlockSpec((RB, D), lambda i: (i, 0)),
            pl.BlockSpec((1, D), lambda i: (0, 0)),
            pl.BlockSpec((1, D), lambda i: (0, 0)),
        ],
        out_specs=pl.BlockSpec((RB, D), lambda i: (i, 0)),
        out_shape=jax.ShapeDtypeStruct((S, D), F32),
    )(op, jnp.asarray(_FC, dtype=BF16), jnp.asarray(_FS, dtype=BF16), x[0],
      gamma.reshape(1, D), beta.reshape(1, D))
    return y[None]
